# initial kernel scaffold (unmeasured)
import jax
import jax.numpy as jnp
from jax import lax
from jax.experimental import pallas as pl
from jax.experimental.pallas import tpu as pltpu

N_DEV = 8
AXIS = "i"
E4M3_MAX = 448.0
NT = 2048


def kernel(x, w_mat):
    K, k_sh = x.shape
    Kw, N = w_mat.shape
    assert K == Kw and K == N_DEV * k_sh
    MB = K // N_DEV
    NTILES = N // NT

    def body(x_ref, w_ref, out_ref, xblk, wbuf, amax_src, amax_buf,
             bsend, brecv, wsem, asend, arecv):
        my = lax.axis_index(AXIS)

        barrier = pltpu.get_barrier_semaphore()
        for off in range(1, N_DEV):
            pl.semaphore_signal(
                barrier, inc=1,
                device_id=((my + off) % N_DEV,),
                device_id_type=pl.DeviceIdType.MESH,
            )
        pl.semaphore_wait(barrier, N_DEV - 1)

        send_descs = []
        for off in range(1, N_DEV):
            p = (my + off) % N_DEV
            d = pltpu.make_async_remote_copy(
                src_ref=x_ref.at[pl.ds(p * MB, MB), :],
                dst_ref=xblk.at[N_DEV - off],
                send_sem=bsend.at[off - 1],
                recv_sem=brecv.at[N_DEV - off],
                device_id=(p,),
                device_id_type=pl.DeviceIdType.MESH,
            )
            d.start()
            send_descs.append(d)

        def wait_block(k):
            pltpu.make_async_remote_copy(
                src_ref=xblk.at[k], dst_ref=xblk.at[k],
                send_sem=bsend.at[0], recv_sem=brecv.at[k],
                device_id=(0,), device_id_type=pl.DeviceIdType.MESH,
            ).wait_recv()

        def w_copy(t):
            k, n = divmod(t, NTILES)
            j = (my + k) % N_DEV
            return pltpu.make_async_copy(
                w_ref.at[pl.ds(j * MB, MB), pl.ds(n * NT, NT)],
                wbuf.at[t % 2],
                wsem.at[t % 2],
            )

        total = N_DEV * NTILES
        w_descs = {0: w_copy(0)}
        w_descs[0].start()
        amaxes = []
        for t in range(total):
            k, n = divmod(t, NTILES)
            if t + 1 < total:
                d2 = w_copy(t + 1)
                d2.start()
                w_descs[t + 1] = d2
            if n == 0 and k > 0:
                wait_block(k)
            w_descs.pop(t).wait()
            xb = xblk[k] if k > 0 else x_ref[pl.ds(my * MB, MB), :]
            part = lax.dot_general(
                xb, wbuf[t % 2], (((1,), (0,)), ((), ())),
                preferred_element_type=jnp.float32,
            )
            ns = pl.ds(n * NT, NT)
            if k == 0:
                out_ref[:, ns] = part
            else:
                acc = out_ref[:, ns] + part
                out_ref[:, ns] = acc
                if k == N_DEV - 1:
                    amaxes.append(jnp.max(jnp.abs(acc)))

        local_amax = amaxes[0]
        for a in amaxes[1:]:
            local_amax = jnp.maximum(local_amax, a)
        amax_src[:, :] = jnp.broadcast_to(local_amax, (8, 128))

        a_descs = []
        for off in range(1, N_DEV):
            p = (my + off) % N_DEV
            d = pltpu.make_async_remote_copy(
                src_ref=amax_src, dst_ref=amax_buf.at[N_DEV - off],
                send_sem=asend.at[off - 1], recv_sem=arecv.at[N_DEV - off],
                device_id=(p,),
                device_id_type=pl.DeviceIdType.MESH,
            )
            d.start()
            a_descs.append(d)
        g = local_amax
        for k in range(1, N_DEV):
            pltpu.make_async_remote_copy(
                src_ref=amax_src, dst_ref=amax_buf.at[k],
                send_sem=asend.at[0], recv_sem=arecv.at[k],
                device_id=(0,), device_id_type=pl.DeviceIdType.MESH,
            ).wait_recv()
            g = jnp.maximum(g, amax_buf[k, 0, 0])

        for d in send_descs:
            d.wait_send()
        for d in a_descs:
            d.wait_send()

        inv = E4M3_MAX / g
        scale = g / E4M3_MAX
        for n in range(NTILES):
            ns = pl.ds(n * NT, NT)
            q = jnp.clip(out_ref[:, ns] * inv, -E4M3_MAX, E4M3_MAX)
            q8 = q.astype(jnp.float8_e4m3fn).astype(jnp.float32)
            out_ref[:, ns] = q8 * scale

    return pl.pallas_call(
        body,
        out_shape=jax.ShapeDtypeStruct((MB, N), jnp.float32),
        in_specs=[
            pl.BlockSpec(memory_space=pltpu.VMEM),
            pl.BlockSpec(memory_space=pltpu.ANY),
        ],
        out_specs=pl.BlockSpec(memory_space=pltpu.VMEM),
        scratch_shapes=[
            pltpu.VMEM((N_DEV, MB, MB), jnp.float32),
            pltpu.VMEM((2, MB, NT), jnp.float32),
            pltpu.VMEM((8, 128), jnp.float32),
            pltpu.VMEM((N_DEV, 8, 128), jnp.float32),
            pltpu.SemaphoreType.DMA((N_DEV - 1,)),
            pltpu.SemaphoreType.DMA((N_DEV,)),
            pltpu.SemaphoreType.DMA((2,)),
            pltpu.SemaphoreType.DMA((N_DEV - 1,)),
            pltpu.SemaphoreType.DMA((N_DEV,)),
        ],
        compiler_params=pltpu.CompilerParams(collective_id=0),
    )(x, w_mat)


# baseline (device time: 151607 ns/iter reference)
import jax
import jax.numpy as jnp
from jax import lax
from jax.experimental import pallas as pl
from jax.experimental.pallas import tpu as pltpu

N_DEV = 8
AXIS = "i"
E4M3_MAX = 448.0
NT = 2048


def kernel(x, w_mat):
    K, k_sh = x.shape
    Kw, N = w_mat.shape
    assert K == Kw and K == N_DEV * k_sh
    MB = K // N_DEV
    NTILES = N // NT

    def body(x_ref, w_ref, out_ref, xblk, wbuf, amax_src, amax_buf,
             bsend, brecv, wsem, asend, arecv):
        my = lax.axis_index(AXIS)

        barrier = pltpu.get_barrier_semaphore()
        for off in range(1, N_DEV):
            pl.semaphore_signal(
                barrier, inc=1,
                device_id=((my + off) % N_DEV,),
                device_id_type=pl.DeviceIdType.MESH,
            )
        pl.semaphore_wait(barrier, N_DEV - 1)

        send_descs = []
        for off in range(1, N_DEV):
            p = (my + off) % N_DEV
            d = pltpu.make_async_remote_copy(
                src_ref=x_ref.at[pl.ds(p * MB, MB), :],
                dst_ref=xblk.at[N_DEV - off],
                send_sem=bsend.at[off - 1],
                recv_sem=brecv.at[N_DEV - off],
                device_id=(p,),
                device_id_type=pl.DeviceIdType.MESH,
            )
            d.start()
            send_descs.append(d)

        def wait_block(k):
            pltpu.make_async_remote_copy(
                src_ref=xblk.at[k], dst_ref=xblk.at[k],
                send_sem=bsend.at[0], recv_sem=brecv.at[k],
                device_id=(0,), device_id_type=pl.DeviceIdType.MESH,
            ).wait_recv()

        def w_copy(t):
            k, n = divmod(t, NTILES)
            j = (my + k) % N_DEV
            return pltpu.make_async_copy(
                w_ref.at[pl.ds(j * MB, MB), pl.ds(n * NT, NT)],
                wbuf.at[t % 2],
                wsem.at[t % 2],
            )

        total = N_DEV * NTILES
        w_descs = {0: w_copy(0)}
        w_descs[0].start()
        amaxes = []
        for t in range(total):
            k, n = divmod(t, NTILES)
            if t + 1 < total:
                d2 = w_copy(t + 1)
                d2.start()
                w_descs[t + 1] = d2
            if n == 0 and k > 0:
                wait_block(k)
            w_descs.pop(t).wait()
            xb = xblk[k] if k > 0 else x_ref[pl.ds(my * MB, MB), :]
            part = lax.dot_general(
                xb, wbuf[t % 2], (((1,), (0,)), ((), ())),
                preferred_element_type=jnp.float32,
            )
            ns = pl.ds(n * NT, NT)
            if k == 0:
                out_ref[:, ns] = part
            else:
                acc = out_ref[:, ns] + part
                out_ref[:, ns] = acc
                if k == N_DEV - 1:
                    amaxes.append(jnp.max(jnp.abs(acc)))

        local_amax = amaxes[0]
        for a in amaxes[1:]:
            local_amax = jnp.maximum(local_amax, a)
        amax_src[:, :] = jnp.broadcast_to(local_amax, (8, 128))

        a_descs = []
        for off in range(1, N_DEV):
            p = (my + off) % N_DEV
            d = pltpu.make_async_remote_copy(
                src_ref=amax_src, dst_ref=amax_buf.at[N_DEV - off],
                send_sem=asend.at[off - 1], recv_sem=arecv.at[N_DEV - off],
                device_id=(p,),
                device_id_type=pl.DeviceIdType.MESH,
            )
            d.start()
            a_descs.append(d)
        g = local_amax
        for k in range(1, N_DEV):
            pltpu.make_async_remote_copy(
                src_ref=amax_src, dst_ref=amax_buf.at[k],
                send_sem=asend.at[0], recv_sem=arecv.at[k],
                device_id=(0,), device_id_type=pl.DeviceIdType.MESH,
            ).wait_recv()
            g = jnp.maximum(g, amax_buf[k, 0, 0])

        for d in send_descs:
            d.wait_send()
        for d in a_descs:
            d.wait_send()

        inv = E4M3_MAX / g
        scale = g / E4M3_MAX
        for n in range(NTILES):
            ns = pl.ds(n * NT, NT)
            q = jnp.clip(out_ref[:, ns] * inv, -E4M3_MAX, E4M3_MAX)
            q8 = q.astype(jnp.float8_e4m3fn).astype(jnp.float32)
            out_ref[:, ns] = q8 * scale

    return pl.pallas_call(
        body,
        out_shape=jax.ShapeDtypeStruct((MB, N), jnp.float32),
        in_specs=[
            pl.BlockSpec(memory_space=pltpu.MemorySpace.VMEM),
            pl.BlockSpec(memory_space=pltpu.MemorySpace.HBM),
        ],
        out_specs=pl.BlockSpec(memory_space=pltpu.MemorySpace.VMEM),
        scratch_shapes=[
            pltpu.VMEM((N_DEV, MB, MB), jnp.float32),
            pltpu.VMEM((2, MB, NT), jnp.float32),
            pltpu.VMEM((8, 128), jnp.float32),
            pltpu.VMEM((N_DEV, 8, 128), jnp.float32),
            pltpu.SemaphoreType.DMA((N_DEV - 1,)),
            pltpu.SemaphoreType.DMA((N_DEV,)),
            pltpu.SemaphoreType.DMA((2,)),
            pltpu.SemaphoreType.DMA((N_DEV - 1,)),
            pltpu.SemaphoreType.DMA((N_DEV,)),
        ],
        compiler_params=pltpu.CompilerParams(
            collective_id=0, vmem_limit_bytes=100 * 1024 * 1024,
        ),
    )(x, w_mat)


# device time: 139972 ns/iter; 1.0831x vs baseline; 1.0831x over previous
import jax
import jax.numpy as jnp
from jax import lax
from jax.experimental import pallas as pl
from jax.experimental.pallas import tpu as pltpu

N_DEV = 8
AXIS = "i"
E4M3_MAX = 448.0
NT = 2048


def kernel(x, w_mat):
    K, k_sh = x.shape
    Kw, N = w_mat.shape
    assert K == Kw and K == N_DEV * k_sh
    MB = K // N_DEV
    NTILES = N // NT

    def body(x_ref, w_ref, out_ref, xblk, wbuf, amax_src, amax_buf,
             bsend, brecv, wsem, asend, arecv):
        my = lax.axis_index(AXIS)

        barrier = pltpu.get_barrier_semaphore()
        for off in range(1, N_DEV):
            pl.semaphore_signal(
                barrier, inc=1,
                device_id=((my + off) % N_DEV,),
                device_id_type=pl.DeviceIdType.MESH,
            )
        pl.semaphore_wait(barrier, N_DEV - 1)

        send_descs = []
        for off in range(1, N_DEV):
            p = (my + off) % N_DEV
            d = pltpu.make_async_remote_copy(
                src_ref=x_ref.at[pl.ds(p * MB, MB), :],
                dst_ref=xblk.at[N_DEV - off],
                send_sem=bsend.at[off - 1],
                recv_sem=brecv.at[N_DEV - off],
                device_id=(p,),
                device_id_type=pl.DeviceIdType.MESH,
            )
            d.start()
            send_descs.append(d)

        def wait_block(k):
            pltpu.make_async_remote_copy(
                src_ref=xblk.at[k], dst_ref=xblk.at[k],
                send_sem=bsend.at[0], recv_sem=brecv.at[k],
                device_id=(0,), device_id_type=pl.DeviceIdType.MESH,
            ).wait_recv()

        HB = MB // 2

        def w_copy(t):
            k, h = divmod(t, 2)
            j = (my + k) % N_DEV
            return pltpu.make_async_copy(
                w_ref.at[pl.ds(j * MB + h * HB, HB), :],
                wbuf.at[t % 2],
                wsem.at[t % 2],
            )

        total = 2 * N_DEV
        w_descs = {0: w_copy(0)}
        w_descs[0].start()
        amaxes = []
        for t in range(total):
            k, h = divmod(t, 2)
            if t + 1 < total:
                d2 = w_copy(t + 1)
                d2.start()
                w_descs[t + 1] = d2
            if k > 0 and h == 0:
                wait_block(k)
            w_descs.pop(t).wait()
            hs = pl.ds(h * HB, HB)
            xb = (xblk[k, :, hs] if k > 0
                  else x_ref[pl.ds(my * MB, MB), hs])
            for n in range(NTILES):
                ns = pl.ds(n * NT, NT)
                part = lax.dot_general(
                    xb, wbuf[t % 2, :, ns], (((1,), (0,)), ((), ())),
                    preferred_element_type=jnp.float32,
                )
                if t == 0:
                    out_ref[:, ns] = part
                else:
                    acc = out_ref[:, ns] + part
                    out_ref[:, ns] = acc
                    if t == total - 1:
                        amaxes.append(jnp.max(jnp.abs(acc)))

        local_amax = amaxes[0]
        for a in amaxes[1:]:
            local_amax = jnp.maximum(local_amax, a)
        amax_src[:, :] = jnp.broadcast_to(local_amax, (8, 128))

        a_descs = []
        for off in range(1, N_DEV):
            p = (my + off) % N_DEV
            d = pltpu.make_async_remote_copy(
                src_ref=amax_src, dst_ref=amax_buf.at[N_DEV - off],
                send_sem=asend.at[off - 1], recv_sem=arecv.at[N_DEV - off],
                device_id=(p,),
                device_id_type=pl.DeviceIdType.MESH,
            )
            d.start()
            a_descs.append(d)
        g = local_amax
        for k in range(1, N_DEV):
            pltpu.make_async_remote_copy(
                src_ref=amax_src, dst_ref=amax_buf.at[k],
                send_sem=asend.at[0], recv_sem=arecv.at[k],
                device_id=(0,), device_id_type=pl.DeviceIdType.MESH,
            ).wait_recv()
            g = jnp.maximum(g, amax_buf[k, 0, 0])

        for d in send_descs:
            d.wait_send()
        for d in a_descs:
            d.wait_send()

        inv = E4M3_MAX / g
        scale = g / E4M3_MAX
        for n in range(NTILES):
            ns = pl.ds(n * NT, NT)
            q = jnp.clip(out_ref[:, ns] * inv, -E4M3_MAX, E4M3_MAX)
            q8 = q.astype(jnp.float8_e4m3fn).astype(jnp.float32)
            out_ref[:, ns] = q8 * scale

    return pl.pallas_call(
        body,
        out_shape=jax.ShapeDtypeStruct((MB, N), jnp.float32),
        in_specs=[
            pl.BlockSpec(memory_space=pltpu.MemorySpace.VMEM),
            pl.BlockSpec(memory_space=pltpu.MemorySpace.HBM),
        ],
        out_specs=pl.BlockSpec(memory_space=pltpu.MemorySpace.VMEM),
        scratch_shapes=[
            pltpu.VMEM((N_DEV, MB, MB), jnp.float32),
            pltpu.VMEM((2, MB // 2, N), jnp.float32),
            pltpu.VMEM((8, 128), jnp.float32),
            pltpu.VMEM((N_DEV, 8, 128), jnp.float32),
            pltpu.SemaphoreType.DMA((N_DEV - 1,)),
            pltpu.SemaphoreType.DMA((N_DEV,)),
            pltpu.SemaphoreType.DMA((2,)),
            pltpu.SemaphoreType.DMA((N_DEV - 1,)),
            pltpu.SemaphoreType.DMA((N_DEV,)),
        ],
        compiler_params=pltpu.CompilerParams(
            collective_id=0, vmem_limit_bytes=63 * 1024 * 1024,
        ),
    )(x, w_mat)


# device time: 87789 ns/iter; 1.7269x vs baseline; 1.5944x over previous
import jax
import jax.numpy as jnp
from jax import lax
from jax.experimental import pallas as pl
from jax.experimental.pallas import tpu as pltpu

N_DEV = 8
AXIS = "i"
E4M3_MAX = 448.0
NT = 2048


def kernel(x, w_mat):
    K, k_sh = x.shape
    Kw, N = w_mat.shape
    assert K == Kw and K == N_DEV * k_sh
    MB = K // N_DEV
    NTILES = N // NT

    def body(x_ref, w_ref, out_ref, xbf, xblk, wbuf, amax_src, amax_buf,
             bsend, brecv, wsem, asend, arecv):
        my = lax.axis_index(AXIS)

        xbf[:, :] = x_ref[:, :].astype(jnp.bfloat16)

        barrier = pltpu.get_barrier_semaphore()
        for off in range(1, N_DEV):
            pl.semaphore_signal(
                barrier, inc=1,
                device_id=((my + off) % N_DEV,),
                device_id_type=pl.DeviceIdType.MESH,
            )
        pl.semaphore_wait(barrier, N_DEV - 1)

        send_descs = []
        for off in range(N_DEV - 1, 0, -1):
            p = (my + off) % N_DEV
            d = pltpu.make_async_remote_copy(
                src_ref=xbf.at[pl.ds(p * MB, MB), :],
                dst_ref=xblk.at[N_DEV - off],
                send_sem=bsend.at[off - 1],
                recv_sem=brecv.at[N_DEV - off],
                device_id=(p,),
                device_id_type=pl.DeviceIdType.MESH,
            )
            d.start()
            send_descs.append(d)

        def wait_block(k):
            pltpu.make_async_remote_copy(
                src_ref=xblk.at[k], dst_ref=xblk.at[k],
                send_sem=bsend.at[0], recv_sem=brecv.at[k],
                device_id=(0,), device_id_type=pl.DeviceIdType.MESH,
            ).wait_recv()

        HB = MB // 2

        def w_copy(t):
            k, h = divmod(t, 2)
            j = (my + k) % N_DEV
            return pltpu.make_async_copy(
                w_ref.at[pl.ds(j * MB + h * HB, HB), :],
                wbuf.at[t % 2],
                wsem.at[t % 2],
            )

        total = 2 * N_DEV
        w_descs = {0: w_copy(0)}
        w_descs[0].start()
        amaxes = []
        for t in range(total):
            k, h = divmod(t, 2)
            if t + 1 < total:
                d2 = w_copy(t + 1)
                d2.start()
                w_descs[t + 1] = d2
            if k > 0 and h == 0:
                wait_block(k)
            w_descs.pop(t).wait()
            hs = pl.ds(h * HB, HB)
            xb = (xblk[k, :, hs].astype(jnp.float32) if k > 0
                  else x_ref[pl.ds(my * MB, MB), hs])
            for n in range(NTILES):
                ns = pl.ds(n * NT, NT)
                part = lax.dot_general(
                    xb, wbuf[t % 2, :, ns], (((1,), (0,)), ((), ())),
                    preferred_element_type=jnp.float32,
                )
                if t == 0:
                    out_ref[:, ns] = part
                else:
                    acc = out_ref[:, ns] + part
                    out_ref[:, ns] = acc
                    if t == total - 1:
                        amaxes.append(jnp.max(jnp.abs(acc)))

        local_amax = amaxes[0]
        for a in amaxes[1:]:
            local_amax = jnp.maximum(local_amax, a)
        amax_src[:, :] = jnp.broadcast_to(local_amax, (8, 128))

        a_descs = []
        for off in range(1, N_DEV):
            p = (my + off) % N_DEV
            d = pltpu.make_async_remote_copy(
                src_ref=amax_src, dst_ref=amax_buf.at[N_DEV - off],
                send_sem=asend.at[off - 1], recv_sem=arecv.at[N_DEV - off],
                device_id=(p,),
                device_id_type=pl.DeviceIdType.MESH,
            )
            d.start()
            a_descs.append(d)
        g = local_amax
        for k in range(1, N_DEV):
            pltpu.make_async_remote_copy(
                src_ref=amax_src, dst_ref=amax_buf.at[k],
                send_sem=asend.at[0], recv_sem=arecv.at[k],
                device_id=(0,), device_id_type=pl.DeviceIdType.MESH,
            ).wait_recv()
            g = jnp.maximum(g, amax_buf[k, 0, 0])

        for d in send_descs:
            d.wait_send()
        for d in a_descs:
            d.wait_send()

        inv = E4M3_MAX / g
        scale = g / E4M3_MAX
        for n in range(NTILES):
            ns = pl.ds(n * NT, NT)
            q = jnp.clip(out_ref[:, ns] * inv, -E4M3_MAX, E4M3_MAX)
            q8 = q.astype(jnp.float8_e4m3fn).astype(jnp.float32)
            out_ref[:, ns] = q8 * scale

    return pl.pallas_call(
        body,
        out_shape=jax.ShapeDtypeStruct((MB, N), jnp.float32),
        in_specs=[
            pl.BlockSpec(memory_space=pltpu.MemorySpace.VMEM),
            pl.BlockSpec(memory_space=pltpu.MemorySpace.HBM),
        ],
        out_specs=pl.BlockSpec(memory_space=pltpu.MemorySpace.VMEM),
        scratch_shapes=[
            pltpu.VMEM((K, k_sh), jnp.bfloat16),
            pltpu.VMEM((N_DEV, MB, MB), jnp.bfloat16),
            pltpu.VMEM((2, MB // 2, N), jnp.float32),
            pltpu.VMEM((8, 128), jnp.float32),
            pltpu.VMEM((N_DEV, 8, 128), jnp.float32),
            pltpu.SemaphoreType.DMA((N_DEV - 1,)),
            pltpu.SemaphoreType.DMA((N_DEV,)),
            pltpu.SemaphoreType.DMA((2,)),
            pltpu.SemaphoreType.DMA((N_DEV - 1,)),
            pltpu.SemaphoreType.DMA((N_DEV,)),
        ],
        compiler_params=pltpu.CompilerParams(
            collective_id=0, vmem_limit_bytes=63 * 1024 * 1024,
        ),
    )(x, w_mat)


# device time: 83229 ns/iter; 1.8216x vs baseline; 1.0548x over previous
import jax
import jax.numpy as jnp
from jax import lax
from jax.experimental import pallas as pl
from jax.experimental.pallas import tpu as pltpu

N_DEV = 8
AXIS = "i"
E4M3_MAX = 448.0
NT = 1024


def kernel(x, w_mat):
    K, k_sh = x.shape
    Kw, N = w_mat.shape
    assert K == Kw and K == N_DEV * k_sh
    MB = K // N_DEV
    NTILES = N // NT

    def body(x_ref, w_ref, out_ref, xbf, xblk, wbuf, stage, amax_src,
             amax_buf, bsend, brecv, wsem, ssem, asend, arecv):
        my = lax.axis_index(AXIS)

        barrier = pltpu.get_barrier_semaphore()
        for off in range(1, N_DEV):
            pl.semaphore_signal(
                barrier, inc=1,
                device_id=((my + off) % N_DEV,),
                device_id_type=pl.DeviceIdType.MESH,
            )

        cp = {}
        for c in range(2):
            cp[c] = pltpu.make_async_copy(
                x_ref.at[pl.ds(c * MB, MB), :], stage.at[c], ssem.at[c])
            cp[c].start()
        for c in range(N_DEV):
            if c + 2 < N_DEV:
                d2 = pltpu.make_async_copy(
                    x_ref.at[pl.ds((c + 2) * MB, MB), :],
                    stage.at[c % 2], ssem.at[c % 2])
            cp.pop(c).wait()
            xbf[pl.ds(c * MB, MB), :] = stage[c % 2].astype(jnp.bfloat16)
            if c + 2 < N_DEV:
                d2.start()
                cp[c + 2] = d2

        pl.semaphore_wait(barrier, N_DEV - 1)

        send_descs = []
        for off in range(N_DEV - 1, 0, -1):
            p = (my + off) % N_DEV
            d = pltpu.make_async_remote_copy(
                src_ref=xbf.at[pl.ds(p * MB, MB), :],
                dst_ref=xblk.at[N_DEV - off],
                send_sem=bsend.at[off - 1],
                recv_sem=brecv.at[N_DEV - off],
                device_id=(p,),
                device_id_type=pl.DeviceIdType.MESH,
            )
            d.start()
            send_descs.append(d)

        def wait_block(k):
            pltpu.make_async_remote_copy(
                src_ref=xblk.at[k], dst_ref=xblk.at[k],
                send_sem=bsend.at[0], recv_sem=brecv.at[k],
                device_id=(0,), device_id_type=pl.DeviceIdType.MESH,
            ).wait_recv()

        def w_copy(k):
            j = (my + k) % N_DEV
            return pltpu.make_async_copy(
                w_ref.at[pl.ds(j * MB, MB), :],
                wbuf.at[k % 2],
                wsem.at[k % 2],
            )

        w_descs = {0: w_copy(0)}
        w_descs[0].start()
        amaxes = []
        for k in range(N_DEV):
            if k + 1 < N_DEV:
                d2 = w_copy(k + 1)
                d2.start()
                w_descs[k + 1] = d2
            if k > 0:
                wait_block(k)
            w_descs.pop(k).wait()
            xb = (xblk[k] if k > 0
                  else xbf[pl.ds(my * MB, MB), :]).astype(jnp.float32)
            for n in range(NTILES):
                ns = pl.ds(n * NT, NT)
                part = lax.dot_general(
                    xb, wbuf[k % 2, :, ns], (((1,), (0,)), ((), ())),
                    preferred_element_type=jnp.float32,
                )
                if k == 0:
                    out_ref[:, ns] = part
                else:
                    acc = out_ref[:, ns] + part
                    out_ref[:, ns] = acc
                    if k == N_DEV - 1:
                        amaxes.append(jnp.max(jnp.abs(acc)))

        local_amax = amaxes[0]
        for a in amaxes[1:]:
            local_amax = jnp.maximum(local_amax, a)
        amax_src[:, :] = jnp.broadcast_to(local_amax, (8, 128))

        a_descs = []
        for off in range(1, N_DEV):
            p = (my + off) % N_DEV
            d = pltpu.make_async_remote_copy(
                src_ref=amax_src, dst_ref=amax_buf.at[N_DEV - off],
                send_sem=asend.at[off - 1], recv_sem=arecv.at[N_DEV - off],
                device_id=(p,),
                device_id_type=pl.DeviceIdType.MESH,
            )
            d.start()
            a_descs.append(d)
        g = local_amax
        for k in range(1, N_DEV):
            pltpu.make_async_remote_copy(
                src_ref=amax_src, dst_ref=amax_buf.at[k],
                send_sem=asend.at[0], recv_sem=arecv.at[k],
                device_id=(0,), device_id_type=pl.DeviceIdType.MESH,
            ).wait_recv()
            g = jnp.maximum(g, amax_buf[k, 0, 0])

        for d in send_descs:
            d.wait_send()
        for d in a_descs:
            d.wait_send()

        inv = E4M3_MAX / g
        scale = g / E4M3_MAX
        for n in range(NTILES):
            ns = pl.ds(n * NT, NT)
            q = jnp.clip(out_ref[:, ns] * inv, -E4M3_MAX, E4M3_MAX)
            q8 = q.astype(jnp.float8_e4m3fn).astype(jnp.float32)
            out_ref[:, ns] = q8 * scale

    return pl.pallas_call(
        body,
        out_shape=jax.ShapeDtypeStruct((MB, N), jnp.float32),
        in_specs=[
            pl.BlockSpec(memory_space=pltpu.MemorySpace.HBM),
            pl.BlockSpec(memory_space=pltpu.MemorySpace.HBM),
        ],
        out_specs=pl.BlockSpec(memory_space=pltpu.MemorySpace.VMEM),
        scratch_shapes=[
            pltpu.VMEM((K, k_sh), jnp.bfloat16),
            pltpu.VMEM((N_DEV, MB, MB), jnp.bfloat16),
            pltpu.VMEM((2, MB, N), jnp.float32),
            pltpu.VMEM((2, MB, k_sh), jnp.float32),
            pltpu.VMEM((8, 128), jnp.float32),
            pltpu.VMEM((N_DEV, 8, 128), jnp.float32),
            pltpu.SemaphoreType.DMA((N_DEV - 1,)),
            pltpu.SemaphoreType.DMA((N_DEV,)),
            pltpu.SemaphoreType.DMA((2,)),
            pltpu.SemaphoreType.DMA((2,)),
            pltpu.SemaphoreType.DMA((N_DEV - 1,)),
            pltpu.SemaphoreType.DMA((N_DEV,)),
        ],
        compiler_params=pltpu.CompilerParams(
            collective_id=0, vmem_limit_bytes=63 * 1024 * 1024,
        ),
    )(x, w_mat)
